# baseline (speedup 1.0000x reference)
import functools

import jax
import jax.numpy as jnp
from jax import lax
from jax.experimental import pallas as pl
from jax.experimental.pallas import tpu as pltpu
from jax.experimental.pallas import tpu_sc as plsc

DIM = 64
GAMMA = 12.0
NC = 2
NS = 16
NW = NC * NS
L = 16

B = 16384
CHUNK = 128
CHUNKS_PER_W = B // NW // CHUNK
N_CHUNKS = B // CHUNK
GROUPS = CHUNK // L


def _body(ent_hbm, rel_hbm, w_hbm, h_hbm, r_hbm, t_hbm, out_hbm,
          hc_v, tc_v, rc_v, he_v, te_v, wv_v, rv_v, out_v, sem):
    wid = lax.axis_index("s") * NC + lax.axis_index("c")
    lane = lax.iota(jnp.int32, L)

    def do_chunk(c, _):
        gc = wid * CHUNKS_PER_W + c
        pltpu.sync_copy(h_hbm.at[gc], hc_v)
        pltpu.sync_copy(t_hbm.at[gc], tc_v)
        pltpu.sync_copy(r_hbm.at[gc], rc_v)
        cp_he = pltpu.async_copy(ent_hbm.at[hc_v], he_v, sem)
        cp_te = pltpu.async_copy(ent_hbm.at[tc_v], te_v, sem)
        cp_w = pltpu.async_copy(w_hbm.at[rc_v], wv_v, sem)
        cp_rv = pltpu.async_copy(rel_hbm.at[rc_v], rv_v, sem)
        cp_he.wait()
        cp_te.wait()
        cp_w.wait()
        cp_rv.wait()

        def group(g, _):
            score_vec = jnp.zeros((L,), jnp.float32)
            for k in range(L):
                i = g * L + k
                w_s = [wv_v[i, pl.ds(j * L, L)] for j in range(DIM // L)]
                e_s = [he_v[i, pl.ds(j * L, L)] - te_v[i, pl.ds(j * L, L)]
                       for j in range(DIM // L)]
                ww = functools.reduce(lambda a, b: a + b,
                                      [w * w for w in w_s])
                ew = functools.reduce(lambda a, b: a + b,
                                      [e * w for e, w in zip(e_s, w_s)])
                s2_v = jnp.maximum(jnp.full((L,), jnp.sum(ww)),
                                   jnp.float32(1e-24))
                alpha = jnp.full((L,), jnp.sum(ew)) / s2_v
                acc = functools.reduce(
                    lambda a, b: a + b,
                    [jnp.abs(e - alpha * w + rv_v[i, pl.ds(j * L, L)])
                     for j, (e, w) in enumerate(zip(e_s, w_s))])
                score = jnp.float32(GAMMA) - jnp.full((L,), jnp.sum(acc))
                score_vec = jnp.where(lane == k, score, score_vec)
            out_v[pl.ds(g * L, L)] = score_vec
            return 0

        lax.fori_loop(0, GROUPS, group, 0)
        pltpu.sync_copy(out_v, out_hbm.at[gc])
        return 0

    lax.fori_loop(0, CHUNKS_PER_W, do_chunk, 0)


@jax.jit
def _transh_sc(ent_weight, rel_full, w_full, h2, r2, t2):
    mesh = plsc.VectorSubcoreMesh(
        core_axis_name="c", subcore_axis_name="s", num_cores=NC, num_subcores=NS
    )
    kfn = pl.kernel(
        _body,
        out_type=jax.ShapeDtypeStruct((N_CHUNKS, CHUNK), jnp.float32),
        mesh=mesh,
        scratch_types=[
            pltpu.VMEM((CHUNK,), jnp.int32),
            pltpu.VMEM((CHUNK,), jnp.int32),
            pltpu.VMEM((CHUNK,), jnp.int32),
            pltpu.VMEM((CHUNK, DIM), jnp.float32),
            pltpu.VMEM((CHUNK, DIM), jnp.float32),
            pltpu.VMEM((CHUNK, DIM), jnp.float32),
            pltpu.VMEM((CHUNK, DIM), jnp.float32),
            pltpu.VMEM((CHUNK,), jnp.float32),
            pltpu.SemaphoreType.DMA,
        ],
        compiler_params=pltpu.CompilerParams(
            needs_layout_passes=False, use_tc_tiling_on_sc=False
        ),
    )
    return kfn(ent_weight, rel_full, w_full, h2, r2, t2)


def kernel(ent_weight, rel_weight, w_weight, h, r, t):
    w_full = jnp.concatenate([w_weight, w_weight], axis=0)
    rel_full = jnp.concatenate([rel_weight, -rel_weight], axis=0)
    h2 = h.reshape(N_CHUNKS, CHUNK)
    r2 = r.reshape(N_CHUNKS, CHUNK)
    t2 = t.reshape(N_CHUNKS, CHUNK)
    out2 = _transh_sc(ent_weight, rel_full, w_full, h2, r2, t2)
    return out2.reshape(B)


# speedup vs baseline: 1.6225x; 1.6225x over previous
import functools

import jax
import jax.numpy as jnp
from jax import lax
from jax.experimental import pallas as pl
from jax.experimental.pallas import tpu as pltpu
from jax.experimental.pallas import tpu_sc as plsc

DIM = 64
GAMMA = 12.0
N_REL = 1000
NC = 2
NS = 16
NW = NC * NS
L = 16

B = 16384
CHUNK = 128
CHUNKS_PER_W = B // NW // CHUNK
N_CHUNKS = B // CHUNK
GROUPS = CHUNK // L
WAVE = 16
WAVES = CHUNK // WAVE


def _body(ent_hbm, rel_hbm, w_hbm, h_hbm, r_hbm, t_hbm, out_hbm,
          hi_v, ti_v, ri_v, he_v, te_v, wv_v, rv_v, out_v, sem):
    wid = lax.axis_index("s") * NC + lax.axis_index("c")
    lane = lax.iota(jnp.int32, L)

    def fire_wave(w):
        h16 = hi_v[pl.ds(w * WAVE, WAVE)]
        t16 = ti_v[pl.ds(w * WAVE, WAVE)]
        r16 = ri_v[pl.ds(w * WAVE, WAVE)]
        rm16 = jnp.where(r16 >= N_REL, r16 - N_REL, r16)
        for k in range(WAVE):
            i = w * WAVE + k
            pltpu.async_copy(ent_hbm.at[h16[k]], he_v.at[i], sem)
            pltpu.async_copy(ent_hbm.at[t16[k]], te_v.at[i], sem)
            pltpu.async_copy(w_hbm.at[rm16[k]], wv_v.at[i], sem)
            pltpu.async_copy(rel_hbm.at[rm16[k]], rv_v.at[i], sem)

    def drain_wave():
        for _ in range(WAVE):
            pltpu.make_async_copy(ent_hbm.at[0], he_v.at[0], sem).wait()
            pltpu.make_async_copy(ent_hbm.at[0], te_v.at[0], sem).wait()
            pltpu.make_async_copy(w_hbm.at[0], wv_v.at[0], sem).wait()
            pltpu.make_async_copy(rel_hbm.at[0], rv_v.at[0], sem).wait()

    def do_chunk(c, _):
        gc = wid * CHUNKS_PER_W + c
        pltpu.sync_copy(h_hbm.at[gc], hi_v)
        pltpu.sync_copy(t_hbm.at[gc], ti_v)
        pltpu.sync_copy(r_hbm.at[gc], ri_v)

        fire_wave(0)

        def wave_step(w, _):
            fire_wave(w)
            drain_wave()
            return 0

        lax.fori_loop(1, WAVES, wave_step, 0)
        drain_wave()

        def group(g, _):
            score_vec = jnp.zeros((L,), jnp.float32)
            rr16 = ri_v[pl.ds(g * L, L)]
            sg16 = jnp.where(rr16 >= N_REL, jnp.float32(-1.0),
                             jnp.float32(1.0))
            for k in range(L):
                i = g * L + k
                sg_v = jnp.full((L,), sg16[k])
                w_s = [wv_v[i, pl.ds(j * L, L)] for j in range(DIM // L)]
                e_s = [he_v[i, pl.ds(j * L, L)] - te_v[i, pl.ds(j * L, L)]
                       for j in range(DIM // L)]
                ww = functools.reduce(lambda a, b: a + b,
                                      [w * w for w in w_s])
                ew = functools.reduce(lambda a, b: a + b,
                                      [e * w for e, w in zip(e_s, w_s)])
                s2_v = jnp.maximum(jnp.full((L,), jnp.sum(ww)),
                                   jnp.float32(1e-24))
                alpha = jnp.full((L,), jnp.sum(ew)) / s2_v
                acc = functools.reduce(
                    lambda a, b: a + b,
                    [jnp.abs(e - alpha * w + sg_v * rv_v[i, pl.ds(j * L, L)])
                     for j, (e, w) in enumerate(zip(e_s, w_s))])
                score = jnp.float32(GAMMA) - jnp.full((L,), jnp.sum(acc))
                score_vec = jnp.where(lane == k, score, score_vec)
            out_v[pl.ds(g * L, L)] = score_vec
            return 0

        lax.fori_loop(0, GROUPS, group, 0)
        pltpu.sync_copy(out_v, out_hbm.at[gc])
        return 0

    lax.fori_loop(0, CHUNKS_PER_W, do_chunk, 0)


@jax.jit
def _transh_sc(ent_weight, rel_weight, w_weight, h2, r2, t2):
    mesh = plsc.VectorSubcoreMesh(
        core_axis_name="c", subcore_axis_name="s", num_cores=NC, num_subcores=NS
    )
    kfn = pl.kernel(
        _body,
        out_type=jax.ShapeDtypeStruct((N_CHUNKS, CHUNK), jnp.float32),
        mesh=mesh,
        scratch_types=[
            pltpu.VMEM((CHUNK,), jnp.int32),
            pltpu.VMEM((CHUNK,), jnp.int32),
            pltpu.VMEM((CHUNK,), jnp.int32),
            pltpu.VMEM((CHUNK, DIM), jnp.float32),
            pltpu.VMEM((CHUNK, DIM), jnp.float32),
            pltpu.VMEM((CHUNK, DIM), jnp.float32),
            pltpu.VMEM((CHUNK, DIM), jnp.float32),
            pltpu.VMEM((CHUNK,), jnp.float32),
            pltpu.SemaphoreType.DMA,
        ],
        compiler_params=pltpu.CompilerParams(
            needs_layout_passes=False, use_tc_tiling_on_sc=True
        ),
    )
    return kfn(ent_weight, rel_weight, w_weight, h2, r2, t2)


def kernel(ent_weight, rel_weight, w_weight, h, r, t):
    h2 = h.reshape(N_CHUNKS, CHUNK)
    r2 = r.reshape(N_CHUNKS, CHUNK)
    t2 = t.reshape(N_CHUNKS, CHUNK)
    out2 = _transh_sc(ent_weight, rel_weight, w_weight, h2, r2, t2)
    return out2.reshape(B)
